# baseline (speedup 1.0000x reference)
import functools

import jax
import jax.numpy as jnp
from jax import lax
from jax.experimental import pallas as pl
from jax.experimental.pallas import tpu as pltpu
from jax.experimental.pallas import tpu_sc as plsc

BETA = 0.2
GAMMA = 0.95
EPS = 1e-6

N = 10000
E = 320000
D = 128

NC = 2
NS = 16
HALF = D // NC
BLK = 128
NBLK = E // BLK
UNIF = NBLK // NS
TAILB = NBLK - UNIF * NS
NBUF = 2
ACC_ROWS = 10240
ZCH = ACC_ROWS // NS
CCH = 80
NCCH = N // CCH


def _rowmax_tc(x):
    def body(x_ref, o_ref):
        o_ref[...] = GAMMA * jnp.max(x_ref[...], axis=1)[None, :]

    return pl.pallas_call(
        body,
        out_shape=jax.ShapeDtypeStruct((1, N), jnp.float32),
    )(x)


@functools.partial(
    pl.kernel,
    out_type=jax.ShapeDtypeStruct((N, D), jnp.float32),
    mesh=plsc.VectorSubcoreMesh(core_axis_name="c", subcore_axis_name="s"),
    compiler_params=pltpu.CompilerParams(needs_layout_passes=False,
                                         use_tc_tiling_on_sc=False),
    scratch_types=[
        pltpu.VMEM((N,), jnp.float32),
        tuple(pltpu.VMEM((2, BLK), jnp.int32) for _ in range(NBUF)),
        pltpu.VMEM((BLK,), jnp.float32),
        tuple(pltpu.VMEM((BLK, D), jnp.float32) for _ in range(NBUF)),
        pltpu.VMEM_SHARED((ACC_ROWS, D), jnp.float32),
        tuple(pltpu.SemaphoreType.DMA for _ in range(NBUF)),
        tuple(pltpu.SemaphoreType.DMA for _ in range(NBUF)),
    ],
)
def _sc_gat(m_hbm, ei_hbm, ea_hbm, x_hbm, out_hbm,
            m_v, idx_v, s_v, ea_v, acc, sem_in, sem_sc):
    c = lax.axis_index("c")
    s = lax.axis_index("s")
    h0 = c * HALF

    def zrow(e, carry):
        z = jnp.zeros((16,), jnp.float32)
        for j in range(D // 16):
            ea_v[0][e, pl.ds(j * 16, 16)] = z
        return carry

    lax.fori_loop(0, BLK, zrow, 0, unroll=4)
    for k in range(ZCH // BLK):
        r0 = s * ZCH + k * BLK
        pltpu.sync_copy(ea_v[0], acc.at[pl.ds(r0, BLK)])

    pltpu.sync_copy(m_hbm.at[0], m_v)
    plsc.subcore_barrier()

    def fire_in(b, q):
        e0 = b * BLK
        pltpu.async_copy(ei_hbm.at[:, pl.ds(e0, BLK)], idx_v[q], sem_in[q])
        pltpu.async_copy(ea_hbm.at[pl.ds(e0, BLK), 0, pl.ds(h0, HALF)],
                         ea_v[q].at[:, pl.ds(0, HALF)], sem_in[q])
        pltpu.async_copy(ea_hbm.at[pl.ds(e0, BLK), 1, pl.ds(h0, HALF)],
                         ea_v[q].at[:, pl.ds(HALF, HALF)], sem_in[q])

    def wait_in(q):
        pltpu.make_async_copy(ei_hbm.at[:, pl.ds(0, BLK)],
                              idx_v[q], sem_in[q]).wait()
        pltpu.make_async_copy(ea_hbm.at[pl.ds(0, BLK), 0, pl.ds(0, HALF)],
                              ea_v[q].at[:, pl.ds(0, HALF)], sem_in[q]).wait()
        pltpu.make_async_copy(ea_hbm.at[pl.ds(0, BLK), 1, pl.ds(0, HALF)],
                              ea_v[q].at[:, pl.ds(HALF, HALF)],
                              sem_in[q]).wait()

    def compute(q):
        for i in range(BLK // 16):
            idx = idx_v[q][0, pl.ds(i * 16, 16)]
            s_v[pl.ds(i * 16, 16)] = plsc.load_gather(m_v, [idx])

        def fma16(g, carry):
            sv16 = s_v[pl.ds(g * 16, 16)]
            for t in range(16):
                e = g * 16 + t
                sv = sv16[t]
                for j in range(HALF // 16):
                    sl = pl.ds(j * 16, 16)
                    sr = pl.ds(HALF + j * 16, 16)
                    ea_v[q][e, sl] = ea_v[q][e, sl] + sv * ea_v[q][e, sr]
            return carry

        lax.fori_loop(0, BLK // 16, fma16, 0)

    def fire_sc(q):
        pltpu.async_copy(ea_v[q], acc.at[idx_v[q].at[1]], sem_sc[q],
                         add=True)

    def wait_sc(q):
        pltpu.make_async_copy(ea_hbm.at[pl.ds(0, BLK), :, pl.ds(0, HALF)],
                              ea_v[q], sem_sc[q]).wait()

    @pl.when(s < TAILB)
    def _tail():
        e0 = s * BLK
        pltpu.sync_copy(ei_hbm.at[:, pl.ds(e0, BLK)], idx_v[0])
        pltpu.sync_copy(ea_hbm.at[pl.ds(e0, BLK), 0, pl.ds(h0, HALF)],
                        ea_v[0].at[:, pl.ds(0, HALF)])
        pltpu.sync_copy(ea_hbm.at[pl.ds(e0, BLK), 1, pl.ds(h0, HALF)],
                        ea_v[0].at[:, pl.ds(HALF, HALF)])
        compute(0)
        pltpu.sync_copy(ea_v[0], acc.at[idx_v[0].at[1]], add=True)

    b0 = TAILB + s * UNIF

    fire_in(b0, 0)
    fire_in(b0 + 1, 1)

    def step(b, q, prefetch):
        wait_in(q)
        compute(q)
        fire_sc(q)
        wait_sc(q)
        if prefetch:
            fire_in(b + NBUF, q)

    def miter(i, carry):
        b = b0 + NBUF * i
        for q in range(NBUF):
            step(b + q, q, True)
        return carry

    lax.fori_loop(0, UNIF // NBUF - 1, miter, 0)

    bl = b0 + UNIF - NBUF
    for q in range(NBUF):
        step(bl + q, q, False)

    plsc.subcore_barrier()

    clo = (s * NCCH) // NS
    chi = ((s + 1) * NCCH) // NS

    def cblk(ch, carry):
        r0 = ch * CCH
        pltpu.sync_copy(acc.at[pl.ds(r0, CCH)], ea_v[0].at[pl.ds(0, CCH)])
        pltpu.sync_copy(x_hbm.at[pl.ds(r0, CCH), pl.ds(h0, HALF)],
                        ea_v[1].at[pl.ds(0, CCH), pl.ds(0, HALF)])

        def crow(e, carry2):
            for j in range(HALF // 16):
                sl = pl.ds(j * 16, 16)
                sr = pl.ds(HALF + j * 16, 16)
                qv = ea_v[0][e, sl]
                ac = ea_v[0][e, sr]
                xv = ea_v[1][e, sl]
                ea_v[1][e, sl] = BETA * xv + (1.0 - BETA) * qv / (ac + EPS)
            return carry2

        lax.fori_loop(0, CCH, crow, 0, unroll=2)
        pltpu.sync_copy(ea_v[1].at[pl.ds(0, CCH), pl.ds(0, HALF)],
                        out_hbm.at[pl.ds(r0, CCH), pl.ds(h0, HALF)])
        return carry

    lax.fori_loop(clo, chi, cblk, 0)


def kernel(x, edge_index, edge_attr):
    m = _rowmax_tc(x)
    return _sc_gat(m, edge_index, edge_attr, x)


# speedup vs baseline: 1.6137x; 1.6137x over previous
import functools

import jax
import jax.numpy as jnp
from jax import lax
from jax.experimental import pallas as pl
from jax.experimental.pallas import tpu as pltpu
from jax.experimental.pallas import tpu_sc as plsc

BETA = 0.2
GAMMA = 0.95
EPS = 1e-6

N = 10000
E = 320000
D = 128

NC = 2
NS = 16
HALF = D // NC
BLK = 128
NBLK = E // BLK
UNIF = NBLK // NS
TAILB = NBLK - UNIF * NS
NBUF = 2
ACC_ROWS = 10240
ZCH = ACC_ROWS // NS
CCH = 80
NCCH = N // CCH


def _rowmax_tc(x):
    def body(x_ref, o_ref):
        o_ref[...] = GAMMA * jnp.max(x_ref[...], axis=1)[None, :]

    return pl.pallas_call(
        body,
        out_shape=jax.ShapeDtypeStruct((1, N), jnp.float32),
    )(x)


@functools.partial(
    pl.kernel,
    out_type=jax.ShapeDtypeStruct((N, D), jnp.float32),
    mesh=plsc.VectorSubcoreMesh(core_axis_name="c", subcore_axis_name="s"),
    compiler_params=pltpu.CompilerParams(needs_layout_passes=False,
                                         use_tc_tiling_on_sc=False),
    scratch_types=[
        pltpu.VMEM((N,), jnp.float32),
        tuple(pltpu.VMEM((2, BLK), jnp.int32) for _ in range(NBUF)),
        pltpu.VMEM((BLK,), jnp.float32),
        tuple(pltpu.VMEM((BLK, 2, HALF), jnp.float32) for _ in range(NBUF)),
        pltpu.VMEM_SHARED((ACC_ROWS, 2, HALF), jnp.float32),
        tuple(pltpu.SemaphoreType.DMA for _ in range(NBUF)),
        tuple(pltpu.SemaphoreType.DMA for _ in range(NBUF)),
    ],
)
def _sc_gat(m_hbm, ei_hbm, ea_hbm, x_hbm, out_hbm,
            m_v, idx_v, s_v, ea_v, acc, sem_in, sem_sc):
    c = lax.axis_index("c")
    s = lax.axis_index("s")
    h0 = c * HALF

    def zrow(e, carry):
        z = jnp.zeros((16,), jnp.float32)
        for p in range(2):
            for j in range(HALF // 16):
                ea_v[0][e, p, pl.ds(j * 16, 16)] = z
        return carry

    lax.fori_loop(0, BLK, zrow, 0, unroll=4)
    for k in range(ZCH // BLK):
        r0 = s * ZCH + k * BLK
        pltpu.sync_copy(ea_v[0], acc.at[pl.ds(r0, BLK)])

    pltpu.sync_copy(m_hbm.at[0], m_v)
    plsc.subcore_barrier()

    def fire_in(b, q):
        e0 = b * BLK
        pltpu.async_copy(ei_hbm.at[:, pl.ds(e0, BLK)], idx_v[q], sem_in[q])
        pltpu.async_copy(ea_hbm.at[pl.ds(e0, BLK), :, pl.ds(h0, HALF)],
                         ea_v[q], sem_in[q])

    def wait_in(q):
        pltpu.make_async_copy(ei_hbm.at[:, pl.ds(0, BLK)],
                              idx_v[q], sem_in[q]).wait()
        pltpu.make_async_copy(ea_hbm.at[pl.ds(0, BLK), :, pl.ds(0, HALF)],
                              ea_v[q], sem_in[q]).wait()

    def compute(q):
        for i in range(BLK // 16):
            idx = idx_v[q][0, pl.ds(i * 16, 16)]
            s_v[pl.ds(i * 16, 16)] = plsc.load_gather(m_v, [idx])

        def fma16(g, carry):
            sv16 = s_v[pl.ds(g * 16, 16)]
            for t in range(16):
                e = g * 16 + t
                sv = sv16[t]
                for j in range(HALF // 16):
                    sl = pl.ds(j * 16, 16)
                    ea_v[q][e, 0, sl] = (ea_v[q][e, 0, sl]
                                         + sv * ea_v[q][e, 1, sl])
            return carry

        lax.fori_loop(0, BLK // 16, fma16, 0)

    def fire_sc(q):
        pltpu.async_copy(ea_v[q], acc.at[idx_v[q].at[1]], sem_sc[q],
                         add=True)

    def wait_sc(q):
        pltpu.make_async_copy(ea_hbm.at[pl.ds(0, BLK), :, pl.ds(0, HALF)],
                              ea_v[q], sem_sc[q]).wait()

    @pl.when(s < TAILB)
    def _tail():
        e0 = s * BLK
        pltpu.sync_copy(ei_hbm.at[:, pl.ds(e0, BLK)], idx_v[0])
        pltpu.sync_copy(ea_hbm.at[pl.ds(e0, BLK), :, pl.ds(h0, HALF)],
                        ea_v[0])
        compute(0)
        pltpu.sync_copy(ea_v[0], acc.at[idx_v[0].at[1]], add=True)

    b0 = TAILB + s * UNIF

    fire_in(b0, 0)
    fire_in(b0 + 1, 1)

    def step(b, q, prefetch):
        wait_in(q)
        compute(q)
        fire_sc(q)
        wait_sc(q)
        if prefetch:
            fire_in(b + NBUF, q)

    def miter(i, carry):
        b = b0 + NBUF * i
        for q in range(NBUF):
            step(b + q, q, True)
        return carry

    lax.fori_loop(0, UNIF // NBUF - 1, miter, 0)

    bl = b0 + UNIF - NBUF
    for q in range(NBUF):
        step(bl + q, q, False)

    plsc.subcore_barrier()

    clo = (s * NCCH) // NS
    chi = ((s + 1) * NCCH) // NS

    def cblk(ch, carry):
        r0 = ch * CCH
        pltpu.sync_copy(acc.at[pl.ds(r0, CCH)], ea_v[0].at[pl.ds(0, CCH)])
        pltpu.sync_copy(x_hbm.at[pl.ds(r0, CCH), pl.ds(h0, HALF)],
                        ea_v[1].at[pl.ds(0, CCH), 0])

        def crow(e, carry2):
            for j in range(HALF // 16):
                sl = pl.ds(j * 16, 16)
                qv = ea_v[0][e, 0, sl]
                ac = ea_v[0][e, 1, sl]
                xv = ea_v[1][e, 0, sl]
                ea_v[1][e, 0, sl] = BETA * xv + (1.0 - BETA) * qv / (ac + EPS)
            return carry2

        lax.fori_loop(0, CCH, crow, 0, unroll=2)
        pltpu.sync_copy(ea_v[1].at[pl.ds(0, CCH), 0],
                        out_hbm.at[pl.ds(r0, CCH), pl.ds(h0, HALF)])
        return carry

    lax.fori_loop(clo, chi, cblk, 0)


def kernel(x, edge_index, edge_attr):
    m = _rowmax_tc(x)
    return _sc_gat(m, edge_index, edge_attr, x)


# speedup vs baseline: 1.7522x; 1.0858x over previous
import functools

import jax
import jax.numpy as jnp
from jax import lax
from jax.experimental import pallas as pl
from jax.experimental.pallas import tpu as pltpu
from jax.experimental.pallas import tpu_sc as plsc

BETA = 0.2
GAMMA = 0.95
EPS = 1e-6

N = 10000
E = 320000
D = 128

NC = 2
NS = 16
HALF = D // NC
BLK = 80
NBLK = E // BLK
PTILE = NBLK // NS
NPIPE = PTILE - 1
NBUF = 3
NROT = NPIPE // NBUF
ACC_ROWS = 10240
ZCH = ACC_ROWS // NS
CCH = 80
NCCH = N // CCH


def _rowmax_tc(x):
    def body(x_ref, o_ref):
        o_ref[...] = GAMMA * jnp.max(x_ref[...], axis=1)[None, :]

    return pl.pallas_call(
        body,
        out_shape=jax.ShapeDtypeStruct((1, N), jnp.float32),
    )(x)


@functools.partial(
    pl.kernel,
    out_type=jax.ShapeDtypeStruct((N, D), jnp.float32),
    mesh=plsc.VectorSubcoreMesh(core_axis_name="c", subcore_axis_name="s"),
    compiler_params=pltpu.CompilerParams(needs_layout_passes=False,
                                         use_tc_tiling_on_sc=False),
    scratch_types=[
        pltpu.VMEM((N,), jnp.float32),
        tuple(pltpu.VMEM((2, BLK), jnp.int32) for _ in range(NBUF)),
        pltpu.VMEM((BLK,), jnp.float32),
        tuple(pltpu.VMEM((BLK, 2, HALF), jnp.float32) for _ in range(NBUF)),
        pltpu.VMEM((CCH, HALF), jnp.float32),
        pltpu.VMEM_SHARED((ACC_ROWS, 2, HALF), jnp.float32),
        tuple(pltpu.SemaphoreType.DMA for _ in range(NBUF)),
        tuple(pltpu.SemaphoreType.DMA for _ in range(NBUF)),
    ],
)
def _sc_gat(m_hbm, ei_hbm, ea_hbm, x_hbm, out_hbm,
            m_v, idx_v, s_v, ea_v, x_v, acc, sem_in, sem_sc):
    c = lax.axis_index("c")
    s = lax.axis_index("s")
    h0 = c * HALF

    def zrow(e, carry):
        z = jnp.zeros((16,), jnp.float32)
        for p in range(2):
            for j in range(HALF // 16):
                ea_v[0][e, p, pl.ds(j * 16, 16)] = z
        return carry

    lax.fori_loop(0, BLK, zrow, 0, unroll=4)
    for k in range(ZCH // BLK):
        r0 = s * ZCH + k * BLK
        pltpu.sync_copy(ea_v[0], acc.at[pl.ds(r0, BLK)])

    pltpu.sync_copy(m_hbm.at[0], m_v)
    plsc.subcore_barrier()

    def fire_in(b, q):
        e0 = b * BLK
        pltpu.async_copy(ei_hbm.at[:, pl.ds(e0, BLK)], idx_v[q], sem_in[q])
        pltpu.async_copy(ea_hbm.at[pl.ds(e0, BLK), :, pl.ds(h0, HALF)],
                         ea_v[q], sem_in[q])

    def wait_in(q):
        pltpu.make_async_copy(ei_hbm.at[:, pl.ds(0, BLK)],
                              idx_v[q], sem_in[q]).wait()
        pltpu.make_async_copy(ea_hbm.at[pl.ds(0, BLK), :, pl.ds(0, HALF)],
                              ea_v[q], sem_in[q]).wait()

    def compute(q):
        for i in range(BLK // 16):
            idx = idx_v[q][0, pl.ds(i * 16, 16)]
            s_v[pl.ds(i * 16, 16)] = plsc.load_gather(m_v, [idx])

        def fma16(g, carry):
            sv16 = s_v[pl.ds(g * 16, 16)]
            for t in range(16):
                e = g * 16 + t
                sv = sv16[t]
                for j in range(HALF // 16):
                    sl = pl.ds(j * 16, 16)
                    ea_v[q][e, 0, sl] = (ea_v[q][e, 0, sl]
                                         + sv * ea_v[q][e, 1, sl])
            return carry

        lax.fori_loop(0, BLK // 16, fma16, 0)

    def fire_sc(q):
        pltpu.async_copy(ea_v[q], acc.at[idx_v[q].at[1]], sem_sc[q],
                         add=True)

    def wait_sc(q):
        pltpu.make_async_copy(ea_hbm.at[pl.ds(0, BLK), :, pl.ds(0, HALF)],
                              ea_v[q], sem_sc[q]).wait()

    b0 = s * PTILE
    pltpu.sync_copy(ei_hbm.at[:, pl.ds(b0 * BLK, BLK)], idx_v[0])
    pltpu.sync_copy(ea_hbm.at[pl.ds(b0 * BLK, BLK), :, pl.ds(h0, HALF)],
                    ea_v[0])
    compute(0)
    pltpu.sync_copy(ea_v[0], acc.at[idx_v[0].at[1]], add=True)

    B0 = b0 + 1

    fire_in(B0, 0)
    fire_in(B0 + 1, 1)

    blast = B0 + NPIPE - 1

    def step(b, q, drain):
        wait_in(q)
        compute(q)
        if drain:
            wait_sc((q + 2) % NBUF)
        fire_in(jnp.minimum(b + 2, blast), (q + 2) % NBUF)
        fire_sc(q)

    step(B0, 0, False)
    step(B0 + 1, 1, True)
    step(B0 + 2, 2, True)

    def miter(i, carry):
        b = B0 + NBUF * i
        for q in range(NBUF):
            step(b + q, q, True)
        return carry

    lax.fori_loop(1, NROT, miter, 0)

    wait_sc(2)
    wait_in(0)
    wait_in(1)

    plsc.subcore_barrier()

    clo = (s * NCCH) // NS
    chi = ((s + 1) * NCCH) // NS

    def cblk(ch, carry):
        r0 = ch * CCH
        pltpu.sync_copy(acc.at[pl.ds(r0, CCH)], ea_v[0])
        pltpu.sync_copy(x_hbm.at[pl.ds(r0, CCH), pl.ds(h0, HALF)], x_v)

        def crow(e, carry2):
            for j in range(HALF // 16):
                sl = pl.ds(j * 16, 16)
                qv = ea_v[0][e, 0, sl]
                ac = ea_v[0][e, 1, sl]
                xv = x_v[e, sl]
                x_v[e, sl] = BETA * xv + (1.0 - BETA) * qv / (ac + EPS)
            return carry2

        lax.fori_loop(0, CCH, crow, 0, unroll=2)
        pltpu.sync_copy(x_v, out_hbm.at[pl.ds(r0, CCH), pl.ds(h0, HALF)])
        return carry

    lax.fori_loop(clo, chi, cblk, 0)


def kernel(x, edge_index, edge_attr):
    m = _rowmax_tc(x)
    return _sc_gat(m, edge_index, edge_attr, x)


# speedup vs baseline: 1.7681x; 1.0091x over previous
import functools

import jax
import jax.numpy as jnp
from jax import lax
from jax.experimental import pallas as pl
from jax.experimental.pallas import tpu as pltpu
from jax.experimental.pallas import tpu_sc as plsc

BETA = 0.2
GAMMA = 0.95
EPS = 1e-6

N = 10000
E = 320000
D = 128

NC = 2
NS = 16
HALF = D // NC
BLK = 80
NBLK = E // BLK
PTILE = NBLK // NS
NPIPE = PTILE - 1
NBUF = 3
NROT = NPIPE // NBUF
ACC_ROWS = 10240
ZCH = ACC_ROWS // NS
CCH = 80
NCCH = N // CCH


def _rowmax_tc(x):
    def body(x_ref, o_ref):
        o_ref[...] = GAMMA * jnp.max(x_ref[...], axis=1)[None, :]

    return pl.pallas_call(
        body,
        out_shape=jax.ShapeDtypeStruct((1, N), jnp.float32),
    )(x)


@functools.partial(
    pl.kernel,
    out_type=jax.ShapeDtypeStruct((N, D), jnp.float32),
    mesh=plsc.VectorSubcoreMesh(core_axis_name="c", subcore_axis_name="s"),
    compiler_params=pltpu.CompilerParams(needs_layout_passes=False,
                                         use_tc_tiling_on_sc=False),
    scratch_types=[
        pltpu.VMEM((N,), jnp.float32),
        tuple(pltpu.VMEM((2, BLK), jnp.int32) for _ in range(NBUF)),
        pltpu.VMEM((BLK,), jnp.float32),
        tuple(pltpu.VMEM((BLK, 2, HALF), jnp.float32) for _ in range(NBUF)),
        pltpu.VMEM((CCH, HALF), jnp.float32),
        pltpu.VMEM_SHARED((ACC_ROWS, 2, HALF), jnp.float32),
        tuple(pltpu.SemaphoreType.DMA for _ in range(NBUF)),
        tuple(pltpu.SemaphoreType.DMA for _ in range(NBUF)),
    ],
)
def _sc_gat(m_hbm, ei_hbm, ea_hbm, x_hbm, out_hbm,
            m_v, idx_v, s_v, ea_v, x_v, acc, sem_in, sem_sc):
    c = lax.axis_index("c")
    s = lax.axis_index("s")
    h0 = c * HALF

    def zrow(e, carry):
        z = jnp.zeros((16,), jnp.float32)
        for p in range(2):
            for j in range(HALF // 16):
                ea_v[0][e, p, pl.ds(j * 16, 16)] = z
        return carry

    lax.fori_loop(0, BLK, zrow, 0, unroll=4)
    for k in range(ZCH // BLK):
        r0 = s * ZCH + k * BLK
        pltpu.async_copy(ea_v[0], acc.at[pl.ds(r0, BLK)], sem_sc[0])

    pltpu.sync_copy(m_hbm.at[0], m_v)
    for k in range(ZCH // BLK):
        r0 = s * ZCH + k * BLK
        pltpu.make_async_copy(ea_v[0], acc.at[pl.ds(r0, BLK)],
                              sem_sc[0]).wait()
    plsc.subcore_barrier()

    def fire_in(b, q):
        e0 = b * BLK
        pltpu.async_copy(ei_hbm.at[:, pl.ds(e0, BLK)], idx_v[q], sem_in[q])
        pltpu.async_copy(ea_hbm.at[pl.ds(e0, BLK), :, pl.ds(h0, HALF)],
                         ea_v[q], sem_in[q])

    def wait_in(q):
        pltpu.make_async_copy(ei_hbm.at[:, pl.ds(0, BLK)],
                              idx_v[q], sem_in[q]).wait()
        pltpu.make_async_copy(ea_hbm.at[pl.ds(0, BLK), :, pl.ds(0, HALF)],
                              ea_v[q], sem_in[q]).wait()

    def compute(q):
        for i in range(BLK // 16):
            idx = idx_v[q][0, pl.ds(i * 16, 16)]
            s_v[pl.ds(i * 16, 16)] = plsc.load_gather(m_v, [idx])

        def fma16(g, carry):
            sv16 = s_v[pl.ds(g * 16, 16)]
            for t in range(16):
                e = g * 16 + t
                sv = sv16[t]
                for j in range(HALF // 16):
                    sl = pl.ds(j * 16, 16)
                    ea_v[q][e, 0, sl] = (ea_v[q][e, 0, sl]
                                         + sv * ea_v[q][e, 1, sl])
            return carry

        lax.fori_loop(0, BLK // 16, fma16, 0)

    def fire_sc(q):
        pltpu.async_copy(ea_v[q], acc.at[idx_v[q].at[1]], sem_sc[q],
                         add=True)

    def wait_sc(q):
        pltpu.make_async_copy(ea_hbm.at[pl.ds(0, BLK), :, pl.ds(0, HALF)],
                              ea_v[q], sem_sc[q]).wait()

    b0 = s * PTILE
    pltpu.sync_copy(ei_hbm.at[:, pl.ds(b0 * BLK, BLK)], idx_v[0])
    pltpu.sync_copy(ea_hbm.at[pl.ds(b0 * BLK, BLK), :, pl.ds(h0, HALF)],
                    ea_v[0])
    compute(0)
    pltpu.sync_copy(ea_v[0], acc.at[idx_v[0].at[1]], add=True)

    B0 = b0 + 1

    fire_in(B0, 0)
    fire_in(B0 + 1, 1)

    blast = B0 + NPIPE - 1

    def step(b, q, drain):
        wait_in(q)
        compute(q)
        if drain:
            wait_sc((q + 2) % NBUF)
        fire_in(jnp.minimum(b + 2, blast), (q + 2) % NBUF)
        fire_sc(q)

    step(B0, 0, False)
    step(B0 + 1, 1, True)
    step(B0 + 2, 2, True)

    def miter(i, carry):
        b = B0 + NBUF * i
        for q in range(NBUF):
            step(b + q, q, True)
        return carry

    lax.fori_loop(1, NROT, miter, 0)

    wait_sc(2)
    wait_in(0)
    wait_in(1)

    plsc.subcore_barrier()

    clo = (s * NCCH) // NS
    chi = ((s + 1) * NCCH) // NS

    def cblk(ch, carry):
        r0 = ch * CCH
        pltpu.sync_copy(acc.at[pl.ds(r0, CCH)], ea_v[0])
        pltpu.sync_copy(x_hbm.at[pl.ds(r0, CCH), pl.ds(h0, HALF)], x_v)

        def crow(e, carry2):
            for j in range(HALF // 16):
                sl = pl.ds(j * 16, 16)
                qv = ea_v[0][e, 0, sl]
                ac = ea_v[0][e, 1, sl]
                xv = x_v[e, sl]
                x_v[e, sl] = BETA * xv + (1.0 - BETA) * qv / (ac + EPS)
            return carry2

        lax.fori_loop(0, CCH, crow, 0, unroll=2)
        pltpu.sync_copy(x_v, out_hbm.at[pl.ds(r0, CCH), pl.ds(h0, HALF)])
        return carry

    lax.fori_loop(clo, chi, cblk, 0)


def kernel(x, edge_index, edge_attr):
    m = _rowmax_tc(x)
    return _sc_gat(m, edge_index, edge_attr, x)
